# baseline (speedup 1.0000x reference)
import functools

import jax
import jax.numpy as jnp
from jax import lax
from jax.experimental import pallas as pl
from jax.experimental.pallas import tpu as pltpu
from jax.experimental.pallas import tpu_sc as plsc

N_FINE, N_HID, N_LAT = 10240, 2560, 640
VARS = 100
LL_IN, LL_EMB = 4, 8
H = 256
HH = 128
LATENT = 128

_PREC = None


def _halves_out(ref, m):
    ref[0] = m[:, :HH]
    ref[1] = m[:, HH:]


def _msg0_body(x_ref, ll_ref, wll_ref, bll_ref, wmx_ref, wml_ref, bm_ref,
               out_ref):
    ll_e = lax.dot_general(ll_ref[...], wll_ref[...],
                           (((0,), (0,)), ((), ())),
                           precision=_PREC) + bll_ref[0]
    m = (lax.dot_general(x_ref[...], wmx_ref[...], (((0,), (0,)), ((), ())),
                         precision=_PREC)
         + jnp.dot(ll_e, wml_ref[...], precision=_PREC) + bm_ref[0])
    _halves_out(out_ref, jnp.maximum(m, 0.0))


def _msg0_call(xf, ll, W_ll, b_ll, Wm, bm):
    R = 1024
    grid = (N_FINE // R,)
    out = pl.pallas_call(
        _msg0_body,
        grid=grid,
        in_specs=[
            pl.BlockSpec((VARS, R), lambda i: (0, i)),
            pl.BlockSpec((LL_IN, R), lambda i: (0, i)),
            pl.BlockSpec((LL_IN, LL_EMB), lambda i: (0, 0)),
            pl.BlockSpec((1, LL_EMB), lambda i: (0, 0)),
            pl.BlockSpec((VARS, H), lambda i: (0, 0)),
            pl.BlockSpec((LL_EMB, H), lambda i: (0, 0)),
            pl.BlockSpec((1, H), lambda i: (0, 0)),
        ],
        out_specs=pl.BlockSpec((2, R, HH), lambda i: (0, i, 0)),
        out_shape=jax.ShapeDtypeStruct((2, N_FINE, HH), jnp.float32),
    )(xf, ll, W_ll, b_ll.reshape(1, -1), Wm[:VARS], Wm[VARS:],
      bm.reshape(1, -1))
    return out.reshape(2 * N_FINE, HH)


def _upd_msg_body(agg_ref, ll_ref, wll_ref, bll_ref, wu0_ref, wu1_ref,
                  wul_ref, bu_ref, wm_ref, bm_ref, out_ref):
    ll_e = lax.dot_general(ll_ref[...], wll_ref[...],
                           (((0,), (0,)), ((), ())),
                           precision=_PREC) + bll_ref[0]
    h = (jnp.dot(agg_ref[0], wu0_ref[...], precision=_PREC)
         + jnp.dot(agg_ref[1], wu1_ref[...], precision=_PREC)
         + jnp.dot(ll_e, wul_ref[...], precision=_PREC) + bu_ref[0])
    h = jnp.maximum(h, 0.0)
    m = jnp.dot(h, wm_ref[...], precision=_PREC) + bm_ref[0]
    _halves_out(out_ref, jnp.maximum(m, 0.0))


def _upd_msg_call(agg, ll, W_ll, b_ll, Wu, bu, Wm, bm, n):
    R = 2048 if n % 2048 == 0 else n
    grid = (n // R,)
    out = pl.pallas_call(
        _upd_msg_body,
        grid=grid,
        in_specs=[
            pl.BlockSpec((2, R, HH), lambda i: (0, i, 0)),
            pl.BlockSpec((LL_IN, R), lambda i: (0, i)),
            pl.BlockSpec((LL_IN, LL_EMB), lambda i: (0, 0)),
            pl.BlockSpec((1, LL_EMB), lambda i: (0, 0)),
            pl.BlockSpec((HH, H), lambda i: (0, 0)),
            pl.BlockSpec((HH, H), lambda i: (0, 0)),
            pl.BlockSpec((LL_EMB, H), lambda i: (0, 0)),
            pl.BlockSpec((1, H), lambda i: (0, 0)),
            pl.BlockSpec((H, H), lambda i: (0, 0)),
            pl.BlockSpec((1, H), lambda i: (0, 0)),
        ],
        out_specs=pl.BlockSpec((2, R, HH), lambda i: (0, i, 0)),
        out_shape=jax.ShapeDtypeStruct((2, n, HH), jnp.float32),
    )(agg.reshape(2, n, HH), ll, W_ll, b_ll.reshape(1, -1), Wu[:HH],
      Wu[HH:H], Wu[H:], bu.reshape(1, -1), Wm, bm.reshape(1, -1))
    return out.reshape(2 * n, HH)


def _latent_body(agg_ref, ll_ref, wll_ref, bll_ref, wu0_ref, wu1_ref,
                 wul_ref, bu_ref, wp_ref, eps_ref, wm_ref, bm_ref, out_ref):
    ll_e = lax.dot_general(ll_ref[...], wll_ref[...],
                           (((0,), (0,)), ((), ())),
                           precision=_PREC) + bll_ref[0]
    h = (jnp.dot(agg_ref[0], wu0_ref[...], precision=_PREC)
         + jnp.dot(agg_ref[1], wu1_ref[...], precision=_PREC)
         + jnp.dot(ll_e, wul_ref[...], precision=_PREC) + bu_ref[0])
    h = jnp.maximum(h, 0.0)
    z = jnp.dot(h, wp_ref[...], precision=_PREC)
    mu = z[:, :LATENT]
    logvar = z[:, LATENT:]
    xs = mu + eps_ref[...] * jnp.exp(logvar * 0.5)
    m = jnp.dot(xs, wm_ref[...], precision=_PREC) + bm_ref[0]
    _halves_out(out_ref, jnp.maximum(m, 0.0))


def _latent_call(agg, ll, W_ll, b_ll, Wu, bu, Wp, eps, Wm, bm):
    n = N_LAT
    out = pl.pallas_call(
        _latent_body,
        out_shape=jax.ShapeDtypeStruct((2, n, HH), jnp.float32),
    )(agg.reshape(2, n, HH), ll, W_ll, b_ll.reshape(1, -1), Wu[:HH],
      Wu[HH:H], Wu[H:], bu.reshape(1, -1), Wp, eps, Wm, bm.reshape(1, -1))
    return out.reshape(2 * n, HH)


def _final_body(agg_ref, ll_ref, wll_ref, bll_ref, wu0_ref, wu1_ref,
                wul_ref, bu_ref, wo_ref, bo_ref, out_ref):
    ll_e = lax.dot_general(ll_ref[...], wll_ref[...],
                           (((0,), (0,)), ((), ())),
                           precision=_PREC) + bll_ref[0]
    h = (jnp.dot(agg_ref[0], wu0_ref[...], precision=_PREC)
         + jnp.dot(agg_ref[1], wu1_ref[...], precision=_PREC)
         + jnp.dot(ll_e, wul_ref[...], precision=_PREC) + bu_ref[0])
    h = jnp.maximum(h, 0.0)
    out_ref[0, 0, 0] = jnp.dot(h, wo_ref[...], precision=_PREC) + bo_ref[0]


def _final_call(agg, ll, W_ll, b_ll, Wu, bu, Wo, bo):
    R = 1024
    grid = (N_FINE // R,)
    return pl.pallas_call(
        _final_body,
        grid=grid,
        in_specs=[
            pl.BlockSpec((2, R, HH), lambda i: (0, i, 0)),
            pl.BlockSpec((LL_IN, R), lambda i: (0, i)),
            pl.BlockSpec((LL_IN, LL_EMB), lambda i: (0, 0)),
            pl.BlockSpec((1, LL_EMB), lambda i: (0, 0)),
            pl.BlockSpec((HH, H), lambda i: (0, 0)),
            pl.BlockSpec((HH, H), lambda i: (0, 0)),
            pl.BlockSpec((LL_EMB, H), lambda i: (0, 0)),
            pl.BlockSpec((1, H), lambda i: (0, 0)),
            pl.BlockSpec((H, VARS), lambda i: (0, 0)),
            pl.BlockSpec((1, VARS), lambda i: (0, 0)),
        ],
        out_specs=pl.BlockSpec((1, 1, 1, R, VARS), lambda i: (0, 0, 0, i, 0)),
        out_shape=jax.ShapeDtypeStruct((1, 1, 1, N_FINE, VARS), jnp.float32),
    )(agg.reshape(2, N_FINE, HH), ll, W_ll, b_ll.reshape(1, -1), Wu[:HH],
      Wu[HH:H], Wu[H:], bu.reshape(1, -1), Wo, bo.reshape(1, -1))


_NC, _NS = 2, 16
_B = 128


def _geom(n_dst, n_edges):
    ept = n_edges // _NS
    budget = 2_000_000 - n_dst * HH - _NS * 2 * ept
    b = _B if budget // (_NS * _B * HH) >= 4 else _B // 2
    nb = max(1, min(6, budget // (_NS * b * HH)))
    return b, ept // b, nb


@functools.cache
def _make_segsum(n_src, n_dst, n_edges):
    ept = n_edges // _NS
    rpt = n_dst // _NS
    mesh = plsc.VectorSubcoreMesh(core_axis_name="c", subcore_axis_name="s")

    _b, steps, nb = _geom(n_dst, n_edges)
    main = steps // nb
    tail = steps % nb

    @functools.partial(
        pl.kernel,
        mesh=mesh,
        out_type=jax.ShapeDtypeStruct((2 * n_dst, HH), jnp.float32),
        scratch_types=[
            pltpu.VMEM((steps, _b), jnp.int32),
            pltpu.VMEM((steps, _b), jnp.int32),
            [pltpu.VMEM((_b, HH), jnp.float32)] * nb,
            [pltpu.SemaphoreType.DMA] * nb,
            pltpu.VMEM_SHARED((n_dst, HH), jnp.float32),
        ],
    )
    def seg(msg_hbm, src_hbm, dst_hbm, out_hbm,
            src_buf, dst_buf, rows, sems, agg_sh):
        c = lax.axis_index("c")
        s = lax.axis_index("s")
        pltpu.sync_copy(src_hbm.at[s], src_buf)
        pltpu.sync_copy(dst_hbm.at[s], dst_buf)

        zv = jnp.zeros((16,), jnp.float32)

        def zrow(i, carry):
            for q in range(HH // 16):
                rows[0][i, pl.ds(q * 16, 16)] = zv
            return carry

        lax.fori_loop(0, _b, zrow, 0)
        off = 0
        while off < rpt:
            size = min(_b, rpt - off)
            pltpu.sync_copy(rows[0].at[pl.ds(0, size)],
                            agg_sh.at[pl.ds(s * rpt + off, size)])
            off += size

        @pl.when(c == 1)
        def _():
            def addrow(i, carry):
                for q in range(_b // 16):
                    sl = pl.ds(q * 16, 16)
                    src_buf[i, sl] = src_buf[i, sl] + n_src
                return carry

            lax.fori_loop(0, steps, addrow, 0)

        plsc.subcore_barrier()

        for b in range(min(nb, steps)):
            pltpu.async_copy(msg_hbm.at[src_buf.at[b]], rows[b], sems[b])

        def ring(j, carry):
            for b in range(nb):
                i = j * nb + b
                pltpu.make_async_copy(msg_hbm.at[src_buf.at[i]], rows[b],
                                      sems[b]).wait()
                pltpu.sync_copy(rows[b], agg_sh.at[dst_buf.at[i]], add=True)

                @pl.when(i + nb < steps)
                def _():
                    pltpu.async_copy(msg_hbm.at[src_buf.at[i + nb]], rows[b],
                                     sems[b])
            return carry

        lax.fori_loop(0, main, ring, 0)
        for b in range(tail):
            i = main * nb + b
            pltpu.make_async_copy(msg_hbm.at[src_buf.at[i]], rows[b],
                                  sems[b]).wait()
            pltpu.sync_copy(rows[b], agg_sh.at[dst_buf.at[i]], add=True)

        plsc.subcore_barrier()
        pltpu.sync_copy(agg_sh.at[pl.ds(s * rpt, rpt)],
                        out_hbm.at[pl.ds(c * n_dst + s * rpt, rpt)])

    return seg


def _segsum(msg2, edges, n_src, n_dst):
    n_edges = edges.shape[1]
    b, steps, _ = _geom(n_dst, n_edges)
    src3 = edges[0].astype(jnp.int32).reshape(_NS, steps, b)
    dst3 = edges[1].astype(jnp.int32).reshape(_NS, steps, b)
    return _make_segsum(n_src, n_dst, n_edges)(msg2, src3, dst3)


def kernel(x, latlons_fine, latlons_hid, latlons_lat, edge_enc0, edge_enc1,
           edge_dec0, edge_dec1, W_ll, b_ll, W_msg_e0, b_msg_e0, W_upd_e0,
           b_upd_e0, W_msg_e1, b_msg_e1, W_upd_e1, b_upd_e1, W_proj_lat,
           W_msg_d0, b_msg_d0, W_upd_d0, b_upd_d0, W_msg_d1, b_msg_d1,
           W_upd_d1, b_upd_d1, W_out, b_out, eps):
    xt = jnp.transpose(x.reshape(N_FINE, VARS))
    llf_t = jnp.transpose(latlons_fine)
    llh_t = jnp.transpose(latlons_hid)
    lll_t = jnp.transpose(latlons_lat)
    msg0 = _msg0_call(xt, llf_t, W_ll, b_ll, W_msg_e0, b_msg_e0)
    agg0 = _segsum(msg0, edge_enc0, N_FINE, N_HID)

    msg1 = _upd_msg_call(agg0, llh_t, W_ll, b_ll, W_upd_e0, b_upd_e0,
                         W_msg_e1, b_msg_e1, N_HID)
    agg1 = _segsum(msg1, edge_enc1, N_HID, N_LAT)

    msg2 = _latent_call(agg1, lll_t, W_ll, b_ll, W_upd_e1, b_upd_e1,
                        W_proj_lat, eps, W_msg_d0, b_msg_d0)
    agg2 = _segsum(msg2, edge_dec0, N_LAT, N_HID)

    msg3 = _upd_msg_call(agg2, llh_t, W_ll, b_ll, W_upd_d0, b_upd_d0,
                         W_msg_d1, b_msg_d1, N_HID)
    agg3 = _segsum(msg3, edge_dec1, N_HID, N_FINE)

    return _final_call(agg3, llf_t, W_ll, b_ll, W_upd_d1, b_upd_d1,
                       W_out, b_out)


# speedup vs baseline: 1.0279x; 1.0279x over previous
import functools

import jax
import jax.numpy as jnp
from jax import lax
from jax.experimental import pallas as pl
from jax.experimental.pallas import tpu as pltpu
from jax.experimental.pallas import tpu_sc as plsc

N_FINE, N_HID, N_LAT = 10240, 2560, 640
VARS = 100
LL_IN, LL_EMB = 4, 8
H = 256
HH = 128
LATENT = 128

_PREC = None


def _halves_out(ref, m):
    ref[0] = m[:, :HH]
    ref[1] = m[:, HH:]


def _msg0_body(x_ref, ll_ref, wll_ref, bll_ref, wmx_ref, wml_ref, bm_ref,
               out_ref):
    ll_e = lax.dot_general(ll_ref[...], wll_ref[...],
                           (((0,), (0,)), ((), ())),
                           precision=_PREC) + bll_ref[0]
    m = (lax.dot_general(x_ref[...], wmx_ref[...], (((0,), (0,)), ((), ())),
                         precision=_PREC)
         + jnp.dot(ll_e, wml_ref[...], precision=_PREC) + bm_ref[0])
    _halves_out(out_ref, jnp.maximum(m, 0.0))


def _msg0_call(xf, ll, W_ll, b_ll, Wm, bm):
    R = 2048
    grid = (N_FINE // R,)
    out = pl.pallas_call(
        _msg0_body,
        grid=grid,
        in_specs=[
            pl.BlockSpec((VARS, R), lambda i: (0, i)),
            pl.BlockSpec((LL_IN, R), lambda i: (0, i)),
            pl.BlockSpec((LL_IN, LL_EMB), lambda i: (0, 0)),
            pl.BlockSpec((1, LL_EMB), lambda i: (0, 0)),
            pl.BlockSpec((VARS, H), lambda i: (0, 0)),
            pl.BlockSpec((LL_EMB, H), lambda i: (0, 0)),
            pl.BlockSpec((1, H), lambda i: (0, 0)),
        ],
        out_specs=pl.BlockSpec((2, R, HH), lambda i: (0, i, 0)),
        out_shape=jax.ShapeDtypeStruct((2, N_FINE, HH), jnp.float32),
    )(xf, ll, W_ll, b_ll.reshape(1, -1), Wm[:VARS], Wm[VARS:],
      bm.reshape(1, -1))
    return out.reshape(2 * N_FINE, HH)


def _upd_msg_body(agg_ref, ll_ref, wll_ref, bll_ref, wu0_ref, wu1_ref,
                  wul_ref, bu_ref, wm_ref, bm_ref, out_ref):
    ll_e = lax.dot_general(ll_ref[...], wll_ref[...],
                           (((0,), (0,)), ((), ())),
                           precision=_PREC) + bll_ref[0]
    h = (jnp.dot(agg_ref[0], wu0_ref[...], precision=_PREC)
         + jnp.dot(agg_ref[1], wu1_ref[...], precision=_PREC)
         + jnp.dot(ll_e, wul_ref[...], precision=_PREC) + bu_ref[0])
    h = jnp.maximum(h, 0.0)
    m = jnp.dot(h, wm_ref[...], precision=_PREC) + bm_ref[0]
    _halves_out(out_ref, jnp.maximum(m, 0.0))


def _upd_msg_call(agg, ll, W_ll, b_ll, Wu, bu, Wm, bm, n):
    R = 2048 if n % 2048 == 0 else n
    grid = (n // R,)
    out = pl.pallas_call(
        _upd_msg_body,
        grid=grid,
        in_specs=[
            pl.BlockSpec((2, R, HH), lambda i: (0, i, 0)),
            pl.BlockSpec((LL_IN, R), lambda i: (0, i)),
            pl.BlockSpec((LL_IN, LL_EMB), lambda i: (0, 0)),
            pl.BlockSpec((1, LL_EMB), lambda i: (0, 0)),
            pl.BlockSpec((HH, H), lambda i: (0, 0)),
            pl.BlockSpec((HH, H), lambda i: (0, 0)),
            pl.BlockSpec((LL_EMB, H), lambda i: (0, 0)),
            pl.BlockSpec((1, H), lambda i: (0, 0)),
            pl.BlockSpec((H, H), lambda i: (0, 0)),
            pl.BlockSpec((1, H), lambda i: (0, 0)),
        ],
        out_specs=pl.BlockSpec((2, R, HH), lambda i: (0, i, 0)),
        out_shape=jax.ShapeDtypeStruct((2, n, HH), jnp.float32),
    )(agg.reshape(2, n, HH), ll, W_ll, b_ll.reshape(1, -1), Wu[:HH],
      Wu[HH:H], Wu[H:], bu.reshape(1, -1), Wm, bm.reshape(1, -1))
    return out.reshape(2 * n, HH)


def _latent_body(agg_ref, ll_ref, wll_ref, bll_ref, wu0_ref, wu1_ref,
                 wul_ref, bu_ref, wp_ref, eps_ref, wm_ref, bm_ref, out_ref):
    ll_e = lax.dot_general(ll_ref[...], wll_ref[...],
                           (((0,), (0,)), ((), ())),
                           precision=_PREC) + bll_ref[0]
    h = (jnp.dot(agg_ref[0], wu0_ref[...], precision=_PREC)
         + jnp.dot(agg_ref[1], wu1_ref[...], precision=_PREC)
         + jnp.dot(ll_e, wul_ref[...], precision=_PREC) + bu_ref[0])
    h = jnp.maximum(h, 0.0)
    z = jnp.dot(h, wp_ref[...], precision=_PREC)
    mu = z[:, :LATENT]
    logvar = z[:, LATENT:]
    xs = mu + eps_ref[...] * jnp.exp(logvar * 0.5)
    m = jnp.dot(xs, wm_ref[...], precision=_PREC) + bm_ref[0]
    _halves_out(out_ref, jnp.maximum(m, 0.0))


def _latent_call(agg, ll, W_ll, b_ll, Wu, bu, Wp, eps, Wm, bm):
    n = N_LAT
    out = pl.pallas_call(
        _latent_body,
        out_shape=jax.ShapeDtypeStruct((2, n, HH), jnp.float32),
    )(agg.reshape(2, n, HH), ll, W_ll, b_ll.reshape(1, -1), Wu[:HH],
      Wu[HH:H], Wu[H:], bu.reshape(1, -1), Wp, eps, Wm, bm.reshape(1, -1))
    return out.reshape(2 * n, HH)


def _final_body(agg_ref, ll_ref, wll_ref, bll_ref, wu0_ref, wu1_ref,
                wul_ref, bu_ref, wo_ref, bo_ref, out_ref):
    ll_e = lax.dot_general(ll_ref[...], wll_ref[...],
                           (((0,), (0,)), ((), ())),
                           precision=_PREC) + bll_ref[0]
    h = (jnp.dot(agg_ref[0], wu0_ref[...], precision=_PREC)
         + jnp.dot(agg_ref[1], wu1_ref[...], precision=_PREC)
         + jnp.dot(ll_e, wul_ref[...], precision=_PREC) + bu_ref[0])
    h = jnp.maximum(h, 0.0)
    out_ref[0, 0, 0] = jnp.dot(h, wo_ref[...], precision=_PREC) + bo_ref[0]


def _final_call(agg, ll, W_ll, b_ll, Wu, bu, Wo, bo):
    R = 2048
    grid = (N_FINE // R,)
    return pl.pallas_call(
        _final_body,
        grid=grid,
        in_specs=[
            pl.BlockSpec((2, R, HH), lambda i: (0, i, 0)),
            pl.BlockSpec((LL_IN, R), lambda i: (0, i)),
            pl.BlockSpec((LL_IN, LL_EMB), lambda i: (0, 0)),
            pl.BlockSpec((1, LL_EMB), lambda i: (0, 0)),
            pl.BlockSpec((HH, H), lambda i: (0, 0)),
            pl.BlockSpec((HH, H), lambda i: (0, 0)),
            pl.BlockSpec((LL_EMB, H), lambda i: (0, 0)),
            pl.BlockSpec((1, H), lambda i: (0, 0)),
            pl.BlockSpec((H, VARS), lambda i: (0, 0)),
            pl.BlockSpec((1, VARS), lambda i: (0, 0)),
        ],
        out_specs=pl.BlockSpec((1, 1, 1, R, VARS), lambda i: (0, 0, 0, i, 0)),
        out_shape=jax.ShapeDtypeStruct((1, 1, 1, N_FINE, VARS), jnp.float32),
    )(agg.reshape(2, N_FINE, HH), ll, W_ll, b_ll.reshape(1, -1), Wu[:HH],
      Wu[HH:H], Wu[H:], bu.reshape(1, -1), Wo, bo.reshape(1, -1))


_NC, _NS = 2, 16
_B = 128


def _geom(n_dst, n_edges):
    ept = n_edges // _NS
    budget = 2_000_000 - n_dst * HH - _NS * 2 * ept
    b = _B if budget // (_NS * _B * HH) >= 4 else _B // 2
    nb = max(1, min(6, budget // (_NS * b * HH)))
    return b, ept // b, nb


@functools.cache
def _make_segsum(n_src, n_dst, n_edges):
    ept = n_edges // _NS
    rpt = n_dst // _NS
    mesh = plsc.VectorSubcoreMesh(core_axis_name="c", subcore_axis_name="s")

    _b, steps, nb = _geom(n_dst, n_edges)
    main = steps // nb
    tail = steps % nb

    @functools.partial(
        pl.kernel,
        mesh=mesh,
        out_type=jax.ShapeDtypeStruct((2 * n_dst, HH), jnp.float32),
        scratch_types=[
            pltpu.VMEM((steps, _b), jnp.int32),
            pltpu.VMEM((steps, _b), jnp.int32),
            [pltpu.VMEM((_b, HH), jnp.float32)] * nb,
            [pltpu.SemaphoreType.DMA] * nb,
            pltpu.VMEM_SHARED((n_dst, HH), jnp.float32),
        ],
    )
    def seg(msg_hbm, src_hbm, dst_hbm, out_hbm,
            src_buf, dst_buf, rows, sems, agg_sh):
        c = lax.axis_index("c")
        s = lax.axis_index("s")
        pltpu.sync_copy(src_hbm.at[s], src_buf)
        pltpu.sync_copy(dst_hbm.at[s], dst_buf)

        zv = jnp.zeros((16,), jnp.float32)

        def zrow(i, carry):
            for q in range(HH // 16):
                rows[0][i, pl.ds(q * 16, 16)] = zv
            return carry

        lax.fori_loop(0, _b, zrow, 0)
        off = 0
        while off < rpt:
            size = min(_b, rpt - off)
            pltpu.sync_copy(rows[0].at[pl.ds(0, size)],
                            agg_sh.at[pl.ds(s * rpt + off, size)])
            off += size

        @pl.when(c == 1)
        def _():
            def addrow(i, carry):
                for q in range(_b // 16):
                    sl = pl.ds(q * 16, 16)
                    src_buf[i, sl] = src_buf[i, sl] + n_src
                return carry

            lax.fori_loop(0, steps, addrow, 0)

        plsc.subcore_barrier()

        for b in range(min(nb, steps)):
            pltpu.async_copy(msg_hbm.at[src_buf.at[b]], rows[b], sems[b])

        def ring(j, carry):
            for b in range(nb):
                i = j * nb + b
                pltpu.make_async_copy(msg_hbm.at[src_buf.at[i]], rows[b],
                                      sems[b]).wait()
                pltpu.sync_copy(rows[b], agg_sh.at[dst_buf.at[i]], add=True)

                @pl.when(i + nb < steps)
                def _():
                    pltpu.async_copy(msg_hbm.at[src_buf.at[i + nb]], rows[b],
                                     sems[b])
            return carry

        lax.fori_loop(0, main, ring, 0)
        for b in range(tail):
            i = main * nb + b
            pltpu.make_async_copy(msg_hbm.at[src_buf.at[i]], rows[b],
                                  sems[b]).wait()
            pltpu.sync_copy(rows[b], agg_sh.at[dst_buf.at[i]], add=True)

        plsc.subcore_barrier()
        pltpu.sync_copy(agg_sh.at[pl.ds(s * rpt, rpt)],
                        out_hbm.at[pl.ds(c * n_dst + s * rpt, rpt)])

    return seg


def _segsum(msg2, edges, n_src, n_dst):
    n_edges = edges.shape[1]
    b, steps, _ = _geom(n_dst, n_edges)
    src3 = edges[0].astype(jnp.int32).reshape(_NS, steps, b)
    dst3 = edges[1].astype(jnp.int32).reshape(_NS, steps, b)
    return _make_segsum(n_src, n_dst, n_edges)(msg2, src3, dst3)


def kernel(x, latlons_fine, latlons_hid, latlons_lat, edge_enc0, edge_enc1,
           edge_dec0, edge_dec1, W_ll, b_ll, W_msg_e0, b_msg_e0, W_upd_e0,
           b_upd_e0, W_msg_e1, b_msg_e1, W_upd_e1, b_upd_e1, W_proj_lat,
           W_msg_d0, b_msg_d0, W_upd_d0, b_upd_d0, W_msg_d1, b_msg_d1,
           W_upd_d1, b_upd_d1, W_out, b_out, eps):
    xt = jnp.transpose(x.reshape(N_FINE, VARS))
    llf_t = jnp.transpose(latlons_fine)
    llh_t = jnp.transpose(latlons_hid)
    lll_t = jnp.transpose(latlons_lat)
    msg0 = _msg0_call(xt, llf_t, W_ll, b_ll, W_msg_e0, b_msg_e0)
    agg0 = _segsum(msg0, edge_enc0, N_FINE, N_HID)

    msg1 = _upd_msg_call(agg0, llh_t, W_ll, b_ll, W_upd_e0, b_upd_e0,
                         W_msg_e1, b_msg_e1, N_HID)
    agg1 = _segsum(msg1, edge_enc1, N_HID, N_LAT)

    msg2 = _latent_call(agg1, lll_t, W_ll, b_ll, W_upd_e1, b_upd_e1,
                        W_proj_lat, eps, W_msg_d0, b_msg_d0)
    agg2 = _segsum(msg2, edge_dec0, N_LAT, N_HID)

    msg3 = _upd_msg_call(agg2, llh_t, W_ll, b_ll, W_upd_d0, b_upd_d0,
                         W_msg_d1, b_msg_d1, N_HID)
    agg3 = _segsum(msg3, edge_dec1, N_HID, N_FINE)

    return _final_call(agg3, llf_t, W_ll, b_ll, W_upd_d1, b_upd_d1,
                       W_out, b_out)


# speedup vs baseline: 1.0474x; 1.0189x over previous
import functools

import jax
import jax.numpy as jnp
from jax import lax
from jax.experimental import pallas as pl
from jax.experimental.pallas import tpu as pltpu
from jax.experimental.pallas import tpu_sc as plsc

N_FINE, N_HID, N_LAT = 10240, 2560, 640
VARS = 100
LL_IN, LL_EMB = 4, 8
H = 256
HH = 128
LATENT = 128

_PREC = None


def _halves_out(ref, m):
    ref[0] = m[:, :HH]
    ref[1] = m[:, HH:]


def _msg0_body(x_ref, ll_ref, wll_ref, bll_ref, wmx_ref, wml_ref, bm_ref,
               out_ref):
    ll_e = lax.dot_general(ll_ref[...], wll_ref[...],
                           (((0,), (0,)), ((), ())),
                           precision=_PREC) + bll_ref[0]
    m = (lax.dot_general(x_ref[...], wmx_ref[...], (((0,), (0,)), ((), ())),
                         precision=_PREC)
         + jnp.dot(ll_e, wml_ref[...], precision=_PREC) + bm_ref[0])
    _halves_out(out_ref, jnp.maximum(m, 0.0))


def _msg0_call(xf, ll, W_ll, b_ll, Wm, bm):
    R = 2048
    grid = (N_FINE // R,)
    out = pl.pallas_call(
        _msg0_body,
        grid=grid,
        in_specs=[
            pl.BlockSpec((VARS, R), lambda i: (0, i)),
            pl.BlockSpec((LL_IN, R), lambda i: (0, i)),
            pl.BlockSpec((LL_IN, LL_EMB), lambda i: (0, 0)),
            pl.BlockSpec((1, LL_EMB), lambda i: (0, 0)),
            pl.BlockSpec((VARS, H), lambda i: (0, 0)),
            pl.BlockSpec((LL_EMB, H), lambda i: (0, 0)),
            pl.BlockSpec((1, H), lambda i: (0, 0)),
        ],
        out_specs=pl.BlockSpec((2, R, HH), lambda i: (0, i, 0)),
        out_shape=jax.ShapeDtypeStruct((2, N_FINE, HH), jnp.float32),
    )(xf, ll, W_ll, b_ll.reshape(1, -1), Wm[:VARS], Wm[VARS:],
      bm.reshape(1, -1))
    return out.reshape(2 * N_FINE, HH)


def _upd_msg_body(agg_ref, ll_ref, wll_ref, bll_ref, wu0_ref, wu1_ref,
                  wul_ref, bu_ref, wm_ref, bm_ref, out_ref):
    ll_e = lax.dot_general(ll_ref[...], wll_ref[...],
                           (((0,), (0,)), ((), ())),
                           precision=_PREC) + bll_ref[0]
    h = (jnp.dot(agg_ref[0], wu0_ref[...], precision=_PREC)
         + jnp.dot(agg_ref[1], wu1_ref[...], precision=_PREC)
         + jnp.dot(ll_e, wul_ref[...], precision=_PREC) + bu_ref[0])
    h = jnp.maximum(h, 0.0)
    m = jnp.dot(h, wm_ref[...], precision=_PREC) + bm_ref[0]
    _halves_out(out_ref, jnp.maximum(m, 0.0))


def _upd_msg_call(agg, ll, W_ll, b_ll, Wu, bu, Wm, bm, n):
    R = 2048 if n % 2048 == 0 else n
    grid = (n // R,)
    out = pl.pallas_call(
        _upd_msg_body,
        grid=grid,
        in_specs=[
            pl.BlockSpec((2, R, HH), lambda i: (0, i, 0)),
            pl.BlockSpec((LL_IN, R), lambda i: (0, i)),
            pl.BlockSpec((LL_IN, LL_EMB), lambda i: (0, 0)),
            pl.BlockSpec((1, LL_EMB), lambda i: (0, 0)),
            pl.BlockSpec((HH, H), lambda i: (0, 0)),
            pl.BlockSpec((HH, H), lambda i: (0, 0)),
            pl.BlockSpec((LL_EMB, H), lambda i: (0, 0)),
            pl.BlockSpec((1, H), lambda i: (0, 0)),
            pl.BlockSpec((H, H), lambda i: (0, 0)),
            pl.BlockSpec((1, H), lambda i: (0, 0)),
        ],
        out_specs=pl.BlockSpec((2, R, HH), lambda i: (0, i, 0)),
        out_shape=jax.ShapeDtypeStruct((2, n, HH), jnp.float32),
    )(agg.reshape(2, n, HH), ll, W_ll, b_ll.reshape(1, -1), Wu[:HH],
      Wu[HH:H], Wu[H:], bu.reshape(1, -1), Wm, bm.reshape(1, -1))
    return out.reshape(2 * n, HH)


def _latent_body(agg_ref, ll_ref, wll_ref, bll_ref, wu0_ref, wu1_ref,
                 wul_ref, bu_ref, wp_ref, eps_ref, wm_ref, bm_ref, out_ref):
    ll_e = lax.dot_general(ll_ref[...], wll_ref[...],
                           (((0,), (0,)), ((), ())),
                           precision=_PREC) + bll_ref[0]
    h = (jnp.dot(agg_ref[0], wu0_ref[...], precision=_PREC)
         + jnp.dot(agg_ref[1], wu1_ref[...], precision=_PREC)
         + jnp.dot(ll_e, wul_ref[...], precision=_PREC) + bu_ref[0])
    h = jnp.maximum(h, 0.0)
    z = jnp.dot(h, wp_ref[...], precision=_PREC)
    mu = z[:, :LATENT]
    logvar = z[:, LATENT:]
    xs = mu + eps_ref[...] * jnp.exp(logvar * 0.5)
    m = jnp.dot(xs, wm_ref[...], precision=_PREC) + bm_ref[0]
    _halves_out(out_ref, jnp.maximum(m, 0.0))


def _latent_call(agg, ll, W_ll, b_ll, Wu, bu, Wp, eps, Wm, bm):
    n = N_LAT
    out = pl.pallas_call(
        _latent_body,
        out_shape=jax.ShapeDtypeStruct((2, n, HH), jnp.float32),
    )(agg.reshape(2, n, HH), ll, W_ll, b_ll.reshape(1, -1), Wu[:HH],
      Wu[HH:H], Wu[H:], bu.reshape(1, -1), Wp, eps, Wm, bm.reshape(1, -1))
    return out.reshape(2 * n, HH)


def _final_body(agg_ref, ll_ref, wll_ref, bll_ref, wu0_ref, wu1_ref,
                wul_ref, bu_ref, wo_ref, bo_ref, out_ref):
    ll_e = lax.dot_general(ll_ref[...], wll_ref[...],
                           (((0,), (0,)), ((), ())),
                           precision=_PREC) + bll_ref[0]
    h = (jnp.dot(agg_ref[0], wu0_ref[...], precision=_PREC)
         + jnp.dot(agg_ref[1], wu1_ref[...], precision=_PREC)
         + jnp.dot(ll_e, wul_ref[...], precision=_PREC) + bu_ref[0])
    h = jnp.maximum(h, 0.0)
    out_ref[0, 0, 0] = jnp.dot(h, wo_ref[...], precision=_PREC) + bo_ref[0]


def _final_call(agg, ll, W_ll, b_ll, Wu, bu, Wo, bo):
    R = 2048
    grid = (N_FINE // R,)
    return pl.pallas_call(
        _final_body,
        grid=grid,
        in_specs=[
            pl.BlockSpec((2, R, HH), lambda i: (0, i, 0)),
            pl.BlockSpec((LL_IN, R), lambda i: (0, i)),
            pl.BlockSpec((LL_IN, LL_EMB), lambda i: (0, 0)),
            pl.BlockSpec((1, LL_EMB), lambda i: (0, 0)),
            pl.BlockSpec((HH, H), lambda i: (0, 0)),
            pl.BlockSpec((HH, H), lambda i: (0, 0)),
            pl.BlockSpec((LL_EMB, H), lambda i: (0, 0)),
            pl.BlockSpec((1, H), lambda i: (0, 0)),
            pl.BlockSpec((H, VARS), lambda i: (0, 0)),
            pl.BlockSpec((1, VARS), lambda i: (0, 0)),
        ],
        out_specs=pl.BlockSpec((1, 1, 1, R, VARS), lambda i: (0, 0, 0, i, 0)),
        out_shape=jax.ShapeDtypeStruct((1, 1, 1, N_FINE, VARS), jnp.float32),
    )(agg.reshape(2, N_FINE, HH), ll, W_ll, b_ll.reshape(1, -1), Wu[:HH],
      Wu[HH:H], Wu[H:], bu.reshape(1, -1), Wo, bo.reshape(1, -1))


_NC, _NS = 2, 16
_B = 128


def _geom(n_dst, n_edges):
    ept = n_edges // _NS
    budget = 2_000_000 - n_dst * HH - _NS * 2 * ept
    b = _B // 2
    nb = max(1, min(8, budget // (_NS * b * HH)))
    return b, ept // b, nb


@functools.cache
def _make_segsum(n_src, n_dst, n_edges):
    ept = n_edges // _NS
    rpt = n_dst // _NS
    mesh = plsc.VectorSubcoreMesh(core_axis_name="c", subcore_axis_name="s")

    _b, steps, nb = _geom(n_dst, n_edges)
    main = steps // nb
    tail = steps % nb

    @functools.partial(
        pl.kernel,
        mesh=mesh,
        out_type=jax.ShapeDtypeStruct((2 * n_dst, HH), jnp.float32),
        scratch_types=[
            pltpu.VMEM((steps, _b), jnp.int32),
            pltpu.VMEM((steps, _b), jnp.int32),
            [pltpu.VMEM((_b, HH), jnp.float32)] * nb,
            [pltpu.SemaphoreType.DMA] * nb,
            pltpu.VMEM_SHARED((n_dst, HH), jnp.float32),
        ],
    )
    def seg(msg_hbm, src_hbm, dst_hbm, out_hbm,
            src_buf, dst_buf, rows, sems, agg_sh):
        c = lax.axis_index("c")
        s = lax.axis_index("s")
        pltpu.sync_copy(src_hbm.at[s], src_buf)
        pltpu.sync_copy(dst_hbm.at[s], dst_buf)

        zv = jnp.zeros((16,), jnp.float32)

        def zrow(i, carry):
            for q in range(HH // 16):
                rows[0][i, pl.ds(q * 16, 16)] = zv
            return carry

        lax.fori_loop(0, _b, zrow, 0)
        off = 0
        while off < rpt:
            size = min(_b, rpt - off)
            pltpu.sync_copy(rows[0].at[pl.ds(0, size)],
                            agg_sh.at[pl.ds(s * rpt + off, size)])
            off += size

        @pl.when(c == 1)
        def _():
            def addrow(i, carry):
                for q in range(_b // 16):
                    sl = pl.ds(q * 16, 16)
                    src_buf[i, sl] = src_buf[i, sl] + n_src
                return carry

            lax.fori_loop(0, steps, addrow, 0)

        plsc.subcore_barrier()

        for b in range(min(nb, steps)):
            pltpu.async_copy(msg_hbm.at[src_buf.at[b]], rows[b], sems[b])

        def ring(j, carry):
            for b in range(nb):
                i = j * nb + b
                pltpu.make_async_copy(msg_hbm.at[src_buf.at[i]], rows[b],
                                      sems[b]).wait()
                pltpu.sync_copy(rows[b], agg_sh.at[dst_buf.at[i]], add=True)

                @pl.when(i + nb < steps)
                def _():
                    pltpu.async_copy(msg_hbm.at[src_buf.at[i + nb]], rows[b],
                                     sems[b])
            return carry

        lax.fori_loop(0, main, ring, 0)
        for b in range(tail):
            i = main * nb + b
            pltpu.make_async_copy(msg_hbm.at[src_buf.at[i]], rows[b],
                                  sems[b]).wait()
            pltpu.sync_copy(rows[b], agg_sh.at[dst_buf.at[i]], add=True)

        plsc.subcore_barrier()
        pltpu.sync_copy(agg_sh.at[pl.ds(s * rpt, rpt)],
                        out_hbm.at[pl.ds(c * n_dst + s * rpt, rpt)])

    return seg


def _segsum(msg2, edges, n_src, n_dst):
    n_edges = edges.shape[1]
    b, steps, _ = _geom(n_dst, n_edges)
    src3 = edges[0].astype(jnp.int32).reshape(_NS, steps, b)
    dst3 = edges[1].astype(jnp.int32).reshape(_NS, steps, b)
    return _make_segsum(n_src, n_dst, n_edges)(msg2, src3, dst3)


def kernel(x, latlons_fine, latlons_hid, latlons_lat, edge_enc0, edge_enc1,
           edge_dec0, edge_dec1, W_ll, b_ll, W_msg_e0, b_msg_e0, W_upd_e0,
           b_upd_e0, W_msg_e1, b_msg_e1, W_upd_e1, b_upd_e1, W_proj_lat,
           W_msg_d0, b_msg_d0, W_upd_d0, b_upd_d0, W_msg_d1, b_msg_d1,
           W_upd_d1, b_upd_d1, W_out, b_out, eps):
    xt = jnp.transpose(x.reshape(N_FINE, VARS))
    llf_t = jnp.transpose(latlons_fine)
    llh_t = jnp.transpose(latlons_hid)
    lll_t = jnp.transpose(latlons_lat)
    msg0 = _msg0_call(xt, llf_t, W_ll, b_ll, W_msg_e0, b_msg_e0)
    agg0 = _segsum(msg0, edge_enc0, N_FINE, N_HID)

    msg1 = _upd_msg_call(agg0, llh_t, W_ll, b_ll, W_upd_e0, b_upd_e0,
                         W_msg_e1, b_msg_e1, N_HID)
    agg1 = _segsum(msg1, edge_enc1, N_HID, N_LAT)

    msg2 = _latent_call(agg1, lll_t, W_ll, b_ll, W_upd_e1, b_upd_e1,
                        W_proj_lat, eps, W_msg_d0, b_msg_d0)
    agg2 = _segsum(msg2, edge_dec0, N_LAT, N_HID)

    msg3 = _upd_msg_call(agg2, llh_t, W_ll, b_ll, W_upd_d0, b_upd_d0,
                         W_msg_d1, b_msg_d1, N_HID)
    agg3 = _segsum(msg3, edge_dec1, N_HID, N_FINE)

    return _final_call(agg3, llf_t, W_ll, b_ll, W_upd_d1, b_upd_d1,
                       W_out, b_out)
